# baseline (speedup 1.0000x reference)
import functools

import jax
import jax.numpy as jnp
from jax import lax
from jax.experimental import pallas as pl
from jax.experimental.pallas import tpu as pltpu
from jax.experimental.pallas import tpu_sc as plsc

_D = 64
_W = 128
_G = 4
_C = _W * _G


@functools.lru_cache(maxsize=None)
def _build(B: int, V: int):
    info = plsc.get_sparse_core_info()
    nc, ns = info.num_cores, info.num_subcores
    nw = nc * ns
    assert B % (nw * _C) == 0
    b_per_w = B // nw
    nchunks = b_per_w // _C
    mesh = plsc.VectorSubcoreMesh(core_axis_name="c", subcore_axis_name="s")

    @functools.partial(
        pl.kernel,
        mesh=mesh,
        out_type=jax.ShapeDtypeStruct((B, _D), jnp.float32),
        scratch_types=[
            pltpu.VMEM((b_per_w,), jnp.int32),
            pltpu.VMEM((_C, _D), jnp.float32),
            pltpu.SemaphoreType.DMA,
        ],
        compiler_params=pltpu.CompilerParams(use_tc_tiling_on_sc=False),
    )
    def emb_kernel(idx_hbm, table_hbm, out_hbm, idx_v, rows_v, gsem):
        wid = lax.axis_index("s") * nc + lax.axis_index("c")
        base = wid * b_per_w
        pltpu.sync_copy(idx_hbm.at[pl.ds(base, b_per_w)], idx_v)

        def chunk(g, carry):
            off = g * _C
            cps = [
                pltpu.async_copy(
                    table_hbm.at[idx_v.at[pl.ds(off + j * _W, _W)]],
                    rows_v.at[pl.ds(j * _W, _W)],
                    gsem,
                )
                for j in range(_G)
            ]
            for cp in cps:
                cp.wait()
            pltpu.sync_copy(rows_v, out_hbm.at[pl.ds(base + off, _C)])
            return carry

        lax.fori_loop(0, nchunks, chunk, 0)

    return emb_kernel


def kernel(token_ids, emb):
    bsz, seq = token_ids.shape
    idx = token_ids.reshape(-1).astype(jnp.int32)
    out = _build(idx.shape[0], emb.shape[0])(idx, emb)
    return out.reshape(bsz, seq, _D)


# speedup vs baseline: 1.0239x; 1.0239x over previous
import functools

import jax
import jax.numpy as jnp
from jax import lax
from jax.experimental import pallas as pl
from jax.experimental.pallas import tpu as pltpu
from jax.experimental.pallas import tpu_sc as plsc

_D = 64
_W = 128
_G = 5
_C = _W * _G


@functools.lru_cache(maxsize=None)
def _build(B: int, V: int):
    info = plsc.get_sparse_core_info()
    nc, ns = info.num_cores, info.num_subcores
    nw = nc * ns
    assert B % (nw * 2 * _C) == 0
    b_per_w = B // nw
    nchunks = b_per_w // _C
    niter = nchunks // 2
    mesh = plsc.VectorSubcoreMesh(core_axis_name="c", subcore_axis_name="s")

    @functools.partial(
        pl.kernel,
        mesh=mesh,
        out_type=jax.ShapeDtypeStruct((B, _D), jnp.float32),
        scratch_types=[
            pltpu.VMEM((b_per_w,), jnp.int32),
            pltpu.VMEM((2, _C, _D), jnp.float32),
            pltpu.SemaphoreType.DMA,
            pltpu.SemaphoreType.DMA,
            pltpu.SemaphoreType.DMA,
            pltpu.SemaphoreType.DMA,
        ],
        compiler_params=pltpu.CompilerParams(use_tc_tiling_on_sc=False),
    )
    def emb_kernel(idx_hbm, table_hbm, out_hbm, idx_v, rows_v, g0, g1, s0, s1):
        gsems = (g0, g1)
        ssems = (s0, s1)
        wid = lax.axis_index("s") * nc + lax.axis_index("c")
        base = wid * b_per_w
        pltpu.sync_copy(idx_hbm.at[pl.ds(base, b_per_w)], idx_v)

        def start_gathers(g, b):
            off = g * _C
            for j in range(_G):
                pltpu.async_copy(
                    table_hbm.at[idx_v.at[pl.ds(off + j * _W, _W)]],
                    rows_v.at[b, pl.ds(j * _W, _W)],
                    gsems[b],
                )

        def drain(sem, b):
            pltpu.make_async_copy(
                out_hbm.at[pl.ds(0, _C)], rows_v.at[b], sem
            ).wait()

        start_gathers(0, 0)
        start_gathers(1, 1)

        def body(i, carry):
            for b in range(2):
                g = 2 * i + b
                drain(gsems[b], b)
                pltpu.async_copy(
                    rows_v.at[b], out_hbm.at[pl.ds(base + g * _C, _C)], ssems[b]
                )

                @pl.when(i < niter - 1)
                def _():
                    drain(ssems[b], b)
                    start_gathers(g + 2, b)

            return carry

        lax.fori_loop(0, niter, body, 0)
        drain(ssems[0], 0)
        drain(ssems[1], 1)

    return emb_kernel


def kernel(token_ids, emb):
    bsz, seq = token_ids.shape
    idx = token_ids.reshape(-1).astype(jnp.int32)
    out = _build(idx.shape[0], emb.shape[0])(idx, emb)
    return out.reshape(bsz, seq, _D)
